# baseline (speedup 1.0000x reference)
import functools

import jax
import jax.numpy as jnp
from jax import lax
from jax.experimental import pallas as pl
from jax.experimental.pallas import tpu as pltpu
from jax.experimental.pallas import tpu_sc as plsc

N_NODES = 10000
N_EDGES = 320000
D = 128

NC = 2
NS = 16
NW = NC * NS

E_PER_W = N_EDGES // NW
CH = 128
NCH = E_PER_W // CH
REM = E_PER_W - NCH * CH
N_PAD = 10240
ROWS_PER_TILE = N_PAD // NS

PIPE_ITERS = 12


def _sc_body(x_hbm, src_hbm, dst_hbm, part_hbm, dstage_hbm,
             accum_sp,
             si0, di0, r0, si1, di1, r1, si2, di2,
             srcr_v, dstr_v, dloc_v,
             sem_i, sem_g, sem_s):
    c = lax.axis_index("c")
    s = lax.axis_index("s")
    wid = c * NS + s
    base = wid * E_PER_W

    zero16 = jnp.zeros((16,), jnp.float32)
    one16 = jnp.ones((16,), jnp.float32)

    def zero_r0(i, carry):
        for k in range(8):
            r0[i, pl.ds(k * 16, 16)] = zero16
        return carry
    lax.fori_loop(0, CH, zero_r0, 0)

    for k in range(5):
        off = pl.multiple_of(s * ROWS_PER_TILE + k * 128, 8)
        pltpu.sync_copy(r0, accum_sp.at[pl.ds(off, 128)])

    def zero_dloc(i, carry):
        dloc_v[pl.ds(pl.multiple_of(i * 16, 16), 16)] = zero16
        return carry
    lax.fori_loop(0, N_PAD // 16, zero_dloc, 0)

    plsc.subcore_barrier()

    def chunk_off(j):
        return pl.multiple_of(jnp.minimum(base + j * CH, N_EDGES - CH), 8)

    def start_idx(j, si, di):
        pltpu.async_copy(src_hbm.at[pl.ds(chunk_off(j), CH)], si, sem_i)
        pltpu.async_copy(dst_hbm.at[pl.ds(chunk_off(j), CH)], di, sem_i)

    def wait_idx(si, di):
        pltpu.make_async_copy(src_hbm.at[pl.ds(0, CH)], si, sem_i).wait()
        pltpu.make_async_copy(dst_hbm.at[pl.ds(0, CH)], di, sem_i).wait()

    def start_gather(si, r):
        pltpu.async_copy(x_hbm.at[si], r, sem_g)

    def wait_gather(r):
        pltpu.make_async_copy(x_hbm.at[pl.ds(0, CH)], r, sem_g).wait()

    def start_scatter(r, di):
        pltpu.async_copy(r, accum_sp.at[di], sem_s, add=True)

    def wait_scatter(r, di):
        pltpu.make_async_copy(r, accum_sp.at[di], sem_s).wait()

    def hist(di):
        for k in range(8):
            idx16 = di[pl.ds(k * 16, 16)]
            plsc.addupdate_scatter(dloc_v, [idx16], one16)

    R = (r0, r1)
    I = ((si0, di0), (si1, di1), (si2, di2))

    def step(j, pr, pi, wait_scat=True, start_nxt=True, idx2=True):
        rcur, rnxt = R[pr], R[1 - pr]
        icur, inxt, inxt2 = I[pi], I[(pi + 1) % 3], I[(pi + 2) % 3]
        wait_gather(rcur)
        if start_nxt:
            wait_idx(*inxt)
        if wait_scat:
            wait_scatter(rnxt, inxt2[1])
        if start_nxt:
            start_gather(inxt[0], rnxt)
        start_scatter(rcur, icur[1])
        if idx2:
            start_idx(j + 2, *inxt2)
        hist(icur[1])

    start_idx(0, si0, di0)
    wait_idx(si0, di0)
    start_gather(si0, r0)
    start_idx(1, si1, di1)
    step(0, 0, 0, wait_scat=False)

    def pipe(m, carry):
        j = 6 * m + 1
        step(j + 0, 1, 1)
        step(j + 1, 0, 2)
        step(j + 2, 1, 0)
        step(j + 3, 0, 1)
        step(j + 4, 1, 2)
        step(j + 5, 0, 0)
        return carry
    lax.fori_loop(0, PIPE_ITERS, pipe, 0)

    step(73, 1, 1)
    step(74, 0, 2)
    step(75, 1, 0)
    step(76, 0, 1, idx2=False)
    step(77, 1, 2, start_nxt=False, idx2=False)
    wait_scatter(r1, di2)

    offr = pl.multiple_of(base + NCH * CH, 8)
    pltpu.sync_copy(src_hbm.at[pl.ds(offr, REM)], srcr_v)
    pltpu.sync_copy(dst_hbm.at[pl.ds(offr, REM)], dstr_v)
    pltpu.async_copy(x_hbm.at[srcr_v], r0.at[pl.ds(0, REM)], sem_g).wait()
    pltpu.sync_copy(r0.at[pl.ds(0, REM)], accum_sp.at[dstr_v], add=True)
    idx16 = dstr_v[pl.ds(0, 16)]
    plsc.addupdate_scatter(dloc_v, [idx16], one16)

    doff = pl.multiple_of(wid * N_PAD, 8)
    pltpu.sync_copy(dloc_v, dstage_hbm.at[pl.ds(doff, N_PAD)])

    plsc.subcore_barrier()

    off = pl.multiple_of(s * ROWS_PER_TILE, 8)
    pltpu.sync_copy(accum_sp.at[pl.ds(off, ROWS_PER_TILE)],
                    part_hbm.at[c, pl.ds(off, ROWS_PER_TILE)])


@functools.partial(
    pl.kernel,
    out_type=[
        jax.ShapeDtypeStruct((NC, N_PAD, D), jnp.float32),
        jax.ShapeDtypeStruct((NW * N_PAD,), jnp.float32),
    ],
    mesh=plsc.VectorSubcoreMesh(core_axis_name="c", subcore_axis_name="s",
                                num_cores=NC),
    compiler_params=pltpu.CompilerParams(needs_layout_passes=False),
    scratch_types=[
        pltpu.VMEM_SHARED((N_PAD, D), jnp.float32),
        pltpu.VMEM((CH,), jnp.int32),
        pltpu.VMEM((CH,), jnp.int32),
        pltpu.VMEM((CH, D), jnp.float32),
        pltpu.VMEM((CH,), jnp.int32),
        pltpu.VMEM((CH,), jnp.int32),
        pltpu.VMEM((CH, D), jnp.float32),
        pltpu.VMEM((CH,), jnp.int32),
        pltpu.VMEM((CH,), jnp.int32),
        pltpu.VMEM((REM,), jnp.int32),
        pltpu.VMEM((REM,), jnp.int32),
        pltpu.VMEM((N_PAD,), jnp.float32),
        pltpu.SemaphoreType.DMA,
        pltpu.SemaphoreType.DMA,
        pltpu.SemaphoreType.DMA,
    ],
)
def _sc_aggregate(x, src, dst, part_out, dstage_out, *scratch):
    _sc_body(x, src, dst, part_out, dstage_out, *scratch)


ROWS_BLK = 512
GRID = N_PAD // ROWS_BLK


def _tc_body(x_ref, p_ref, dg_ref, ws_ref, wn_ref, bs_ref, bn_ref, o_ref):
    dsum = jnp.sum(dg_ref[...], axis=0, keepdims=True)
    inv_row = 1.0 / jnp.maximum(dsum, 1.0)
    inv_col = jnp.reshape(inv_row, (ROWS_BLK, 1))
    psum = p_ref[0] + p_ref[1]
    dn = (((1,), (1,)), ((), ()))
    o_ref[...] = (
        lax.dot_general(x_ref[...], ws_ref[...], dn,
                        preferred_element_type=jnp.float32)
        + lax.dot_general(psum, wn_ref[...], dn,
                          preferred_element_type=jnp.float32) * inv_col
        + bs_ref[...] + bn_ref[...]
    )


_tc_combine = pl.pallas_call(
    _tc_body,
    grid=(GRID,),
    in_specs=[
        pl.BlockSpec((ROWS_BLK, D), lambda i: (i, 0)),
        pl.BlockSpec((NC, ROWS_BLK, D), lambda i: (0, i, 0)),
        pl.BlockSpec((NW, ROWS_BLK), lambda i: (0, i)),
        pl.BlockSpec((D, D), lambda i: (0, 0)),
        pl.BlockSpec((D, D), lambda i: (0, 0)),
        pl.BlockSpec((1, D), lambda i: (0, 0)),
        pl.BlockSpec((1, D), lambda i: (0, 0)),
    ],
    out_specs=pl.BlockSpec((ROWS_BLK, D), lambda i: (i, 0)),
    out_shape=jax.ShapeDtypeStruct((N_NODES, D), jnp.float32),
)


def kernel(x, edge_index, W_self, b_self, W_neigh, b_neigh):
    src = edge_index[0]
    dst = edge_index[1]
    parts, dstage = _sc_aggregate(x, src, dst)
    out = _tc_combine(
        x, parts, dstage.reshape(NW, N_PAD),
        W_self, W_neigh,
        b_self[None, :], b_neigh[None, :],
    )
    return out


# speedup vs baseline: 1.1054x; 1.1054x over previous
import functools

import jax
import jax.numpy as jnp
from jax import lax
from jax.experimental import pallas as pl
from jax.experimental.pallas import tpu as pltpu
from jax.experimental.pallas import tpu_sc as plsc

N_NODES = 10000
N_EDGES = 320000
D = 128

NC = 2
NS = 16
NW = NC * NS

CH = 128
NCHUNKS = N_EDGES // CH
NCH = NCHUNKS // NW
NXTRA = NCHUNKS - NCH * NW
N_PAD = 10240
ROWS_PER_TILE = N_PAD // NS

PIPE_ITERS = 12


def _sc_body(x_hbm, ei_hbm, part_hbm, dstage_hbm,
             accum_sp,
             ix0, r0, ix1, r1, ix2,
             ixr_v, dloc_v,
             sem_i, sem_g, sem_s):
    c = lax.axis_index("c")
    s = lax.axis_index("s")
    wid = c * NS + s

    zero16 = jnp.zeros((16,), jnp.float32)
    one16 = jnp.ones((16,), jnp.float32)

    def zero_r0(i, carry):
        for k in range(8):
            r0[i, pl.ds(k * 16, 16)] = zero16
        return carry
    lax.fori_loop(0, CH, zero_r0, 0)

    for k in range(5):
        off = pl.multiple_of(s * ROWS_PER_TILE + k * 128, 8)
        pltpu.sync_copy(r0, accum_sp.at[pl.ds(off, 128)])

    def zero_dloc(i, carry):
        dloc_v[pl.ds(pl.multiple_of(i * 16, 16), 16)] = zero16
        return carry
    lax.fori_loop(0, N_PAD // 16, zero_dloc, 0)

    plsc.subcore_barrier()

    def chunk_off(j):
        return pl.multiple_of((wid + NW * j) * CH, 128)

    def start_idx(j, ix):
        pltpu.async_copy(ei_hbm.at[:, pl.ds(chunk_off(j), CH)], ix, sem_i)

    def wait_idx(ix):
        pltpu.make_async_copy(ei_hbm.at[:, pl.ds(0, CH)], ix, sem_i).wait()

    def start_gather(ix, r):
        pltpu.async_copy(x_hbm.at[ix.at[0]], r, sem_g)

    def wait_gather(r):
        pltpu.make_async_copy(x_hbm.at[pl.ds(0, CH)], r, sem_g).wait()

    def start_scatter(r, ix):
        pltpu.async_copy(r, accum_sp.at[ix.at[1]], sem_s, add=True)

    def wait_scatter(r, ix):
        pltpu.make_async_copy(r, accum_sp.at[ix.at[1]], sem_s).wait()

    def hist(ix):
        for k in range(8):
            idx16 = ix[1, pl.ds(k * 16, 16)]
            plsc.addupdate_scatter(dloc_v, [idx16], one16)

    R = (r0, r1)
    I = (ix0, ix1, ix2)

    def step(j, pr, pi, wait_scat=True, start_nxt=True, idx2=True):
        rcur, rnxt = R[pr], R[1 - pr]
        icur, inxt, inxt2 = I[pi], I[(pi + 1) % 3], I[(pi + 2) % 3]
        wait_gather(rcur)
        if start_nxt:
            wait_idx(inxt)
        if wait_scat:
            wait_scatter(rnxt, inxt2)
        if start_nxt:
            start_gather(inxt, rnxt)
        start_scatter(rcur, icur)
        if idx2:
            start_idx(j + 2, inxt2)
        hist(icur)

    start_idx(0, ix0)
    wait_idx(ix0)
    start_gather(ix0, r0)
    start_idx(1, ix1)
    step(0, 0, 0, wait_scat=False)

    def pipe(m, carry):
        j = 6 * m + 1
        step(j + 0, 1, 1)
        step(j + 1, 0, 2)
        step(j + 2, 1, 0)
        step(j + 3, 0, 1)
        step(j + 4, 1, 2)
        step(j + 5, 0, 0)
        return carry
    lax.fori_loop(0, PIPE_ITERS, pipe, 0)

    step(73, 1, 1)
    step(74, 0, 2)
    step(75, 1, 0)
    step(76, 0, 1, idx2=False)
    step(77, 1, 2, start_nxt=False, idx2=False)
    wait_scatter(r1, ix2)

    @pl.when(wid < NXTRA)
    def _():
        offr = pl.multiple_of((NCH * NW + wid) * CH, 128)
        pltpu.sync_copy(ei_hbm.at[:, pl.ds(offr, CH)], ixr_v)
        pltpu.async_copy(x_hbm.at[ixr_v.at[0]], r0, sem_g).wait()
        pltpu.sync_copy(r0, accum_sp.at[ixr_v.at[1]], add=True)
        for k in range(8):
            idx16 = ixr_v[1, pl.ds(k * 16, 16)]
            plsc.addupdate_scatter(dloc_v, [idx16], one16)

    doff = pl.multiple_of(wid * N_PAD, 8)
    pltpu.sync_copy(dloc_v, dstage_hbm.at[pl.ds(doff, N_PAD)])

    plsc.subcore_barrier()

    off = pl.multiple_of(s * ROWS_PER_TILE, 8)
    pltpu.sync_copy(accum_sp.at[pl.ds(off, ROWS_PER_TILE)],
                    part_hbm.at[c, pl.ds(off, ROWS_PER_TILE)])


@functools.partial(
    pl.kernel,
    out_type=[
        jax.ShapeDtypeStruct((NC, N_PAD, D), jnp.float32),
        jax.ShapeDtypeStruct((NW * N_PAD,), jnp.float32),
    ],
    mesh=plsc.VectorSubcoreMesh(core_axis_name="c", subcore_axis_name="s",
                                num_cores=NC),
    compiler_params=pltpu.CompilerParams(needs_layout_passes=False),
    scratch_types=[
        pltpu.VMEM_SHARED((N_PAD, D), jnp.float32),
        pltpu.VMEM((2, CH), jnp.int32),
        pltpu.VMEM((CH, D), jnp.float32),
        pltpu.VMEM((2, CH), jnp.int32),
        pltpu.VMEM((CH, D), jnp.float32),
        pltpu.VMEM((2, CH), jnp.int32),
        pltpu.VMEM((2, CH), jnp.int32),
        pltpu.VMEM((N_PAD,), jnp.float32),
        pltpu.SemaphoreType.DMA,
        pltpu.SemaphoreType.DMA,
        pltpu.SemaphoreType.DMA,
    ],
)
def _sc_aggregate(x, edge_index, part_out, dstage_out, *scratch):
    _sc_body(x, edge_index, part_out, dstage_out, *scratch)


ROWS_BLK = 1024
GRID = N_PAD // ROWS_BLK


def _tc_body(x_ref, p_ref, dg_ref, ws_ref, wn_ref, bs_ref, bn_ref, o_ref):
    dsum = jnp.sum(dg_ref[...], axis=0, keepdims=True)
    inv_row = 1.0 / jnp.maximum(dsum, 1.0)
    inv_col = jnp.reshape(inv_row, (ROWS_BLK, 1))
    psum = p_ref[0] + p_ref[1]
    dn = (((1,), (1,)), ((), ()))
    o_ref[...] = (
        lax.dot_general(x_ref[...], ws_ref[...], dn,
                        preferred_element_type=jnp.float32)
        + lax.dot_general(psum, wn_ref[...], dn,
                          preferred_element_type=jnp.float32) * inv_col
        + bs_ref[...] + bn_ref[...]
    )


_tc_combine = pl.pallas_call(
    _tc_body,
    grid=(GRID,),
    in_specs=[
        pl.BlockSpec((ROWS_BLK, D), lambda i: (i, 0)),
        pl.BlockSpec((NC, ROWS_BLK, D), lambda i: (0, i, 0)),
        pl.BlockSpec((NW, ROWS_BLK), lambda i: (0, i)),
        pl.BlockSpec((D, D), lambda i: (0, 0)),
        pl.BlockSpec((D, D), lambda i: (0, 0)),
        pl.BlockSpec((1, D), lambda i: (0, 0)),
        pl.BlockSpec((1, D), lambda i: (0, 0)),
    ],
    out_specs=pl.BlockSpec((ROWS_BLK, D), lambda i: (i, 0)),
    out_shape=jax.ShapeDtypeStruct((N_NODES, D), jnp.float32),
)


def kernel(x, edge_index, W_self, b_self, W_neigh, b_neigh):
    parts, dstage = _sc_aggregate(x, edge_index)
    out = _tc_combine(
        x, parts, dstage.reshape(NW, N_PAD),
        W_self, W_neigh,
        b_self[None, :], b_neigh[None, :],
    )
    return out


# speedup vs baseline: 1.1152x; 1.0089x over previous
import functools

import jax
import jax.numpy as jnp
from jax import lax
from jax.experimental import pallas as pl
from jax.experimental.pallas import tpu as pltpu
from jax.experimental.pallas import tpu_sc as plsc

N_NODES = 10000
N_EDGES = 320000
D = 128

NC = 2
NS = 16
NW = NC * NS

CH = 128
NCHUNKS = N_EDGES // CH
NCH = NCHUNKS // NW
NXTRA = NCHUNKS - NCH * NW
N_PAD = 10240
ROWS_PER_TILE = N_PAD // NS

PIPE_ITERS = 12


def _sc_body(x_hbm, ei_hbm, part_hbm, dstage_hbm,
             accum_sp,
             ix0, r0, ix1, r1, ix2,
             ixr_v, dloc_v,
             sem_i, sem_g, sem_s):
    c = lax.axis_index("c")
    s = lax.axis_index("s")
    wid = c * NS + s

    zero16 = jnp.zeros((16,), jnp.float32)
    one16 = jnp.ones((16,), jnp.float32)

    def zero_r0(i, carry):
        for k in range(8):
            r0[i, pl.ds(k * 16, 16)] = zero16
        return carry
    lax.fori_loop(0, CH, zero_r0, 0)

    for k in range(5):
        off = pl.multiple_of(s * ROWS_PER_TILE + k * 128, 8)
        pltpu.sync_copy(r0, accum_sp.at[pl.ds(off, 128)])

    def zero_dloc(i, carry):
        dloc_v[pl.ds(pl.multiple_of(i * 16, 16), 16)] = zero16
        return carry
    lax.fori_loop(0, N_PAD // 16, zero_dloc, 0)

    plsc.subcore_barrier()

    def chunk_off(j):
        return pl.multiple_of((wid + NW * j) * CH, 128)

    def start_idx(j, ix):
        pltpu.async_copy(ei_hbm.at[:, pl.ds(chunk_off(j), CH)], ix, sem_i)

    def wait_idx(ix):
        pltpu.make_async_copy(ei_hbm.at[:, pl.ds(0, CH)], ix, sem_i).wait()

    def start_gather(ix, r):
        pltpu.async_copy(x_hbm.at[ix.at[0]], r, sem_g)

    def wait_gather(r):
        pltpu.make_async_copy(x_hbm.at[pl.ds(0, CH)], r, sem_g).wait()

    def start_scatter(r, ix):
        pltpu.async_copy(r, accum_sp.at[ix.at[1]], sem_s, add=True)

    def wait_scatter(r, ix):
        pltpu.make_async_copy(r, accum_sp.at[ix.at[1]], sem_s).wait()

    def hist(ix):
        for k in range(8):
            idx16 = ix[1, pl.ds(k * 16, 16)]
            plsc.addupdate_scatter(dloc_v, [idx16], one16)

    R = (r0, r1)
    I = (ix0, ix1, ix2)

    def step(j, pr, pi, wait_scat=True, start_nxt=True, idx2=True):
        rcur, rnxt = R[pr], R[1 - pr]
        icur, inxt, inxt2 = I[pi], I[(pi + 1) % 3], I[(pi + 2) % 3]
        wait_gather(rcur)
        if start_nxt:
            wait_idx(inxt)
        if wait_scat:
            wait_scatter(rnxt, inxt2)
        if start_nxt:
            start_gather(inxt, rnxt)
        start_scatter(rcur, icur)
        if idx2:
            start_idx(j + 2, inxt2)
        hist(icur)

    start_idx(0, ix0)
    wait_idx(ix0)
    start_gather(ix0, r0)
    start_idx(1, ix1)
    step(0, 0, 0, wait_scat=False)

    def pipe(m, carry):
        j = 6 * m + 1
        step(j + 0, 1, 1)
        step(j + 1, 0, 2)
        step(j + 2, 1, 0)
        step(j + 3, 0, 1)
        step(j + 4, 1, 2)
        step(j + 5, 0, 0)
        return carry
    lax.fori_loop(0, PIPE_ITERS, pipe, 0)

    step(73, 1, 1)
    step(74, 0, 2)
    step(75, 1, 0)
    step(76, 0, 1, idx2=False)
    step(77, 1, 2, start_nxt=False, idx2=False)
    wait_scatter(r1, ix2)

    @pl.when(wid < NXTRA)
    def _():
        offr = pl.multiple_of((NCH * NW + wid) * CH, 128)
        pltpu.sync_copy(ei_hbm.at[:, pl.ds(offr, CH)], ixr_v)
        pltpu.async_copy(x_hbm.at[ixr_v.at[0]], r0, sem_g).wait()
        pltpu.sync_copy(r0, accum_sp.at[ixr_v.at[1]], add=True)
        for k in range(8):
            idx16 = ixr_v[1, pl.ds(k * 16, 16)]
            plsc.addupdate_scatter(dloc_v, [idx16], one16)

    doff = pl.multiple_of(wid * N_PAD, 8)
    pltpu.sync_copy(dloc_v, dstage_hbm.at[pl.ds(doff, N_PAD)])

    plsc.subcore_barrier()

    off = pl.multiple_of(s * ROWS_PER_TILE, 8)
    pltpu.sync_copy(accum_sp.at[pl.ds(off, ROWS_PER_TILE)],
                    part_hbm.at[c, pl.ds(off, ROWS_PER_TILE)])


@functools.partial(
    pl.kernel,
    out_type=[
        jax.ShapeDtypeStruct((NC, N_PAD, D), jnp.float32),
        jax.ShapeDtypeStruct((NW * N_PAD,), jnp.float32),
    ],
    mesh=plsc.VectorSubcoreMesh(core_axis_name="c", subcore_axis_name="s",
                                num_cores=NC),
    compiler_params=pltpu.CompilerParams(needs_layout_passes=False),
    scratch_types=[
        pltpu.VMEM_SHARED((N_PAD, D), jnp.float32),
        pltpu.VMEM((2, CH), jnp.int32),
        pltpu.VMEM((CH, D), jnp.float32),
        pltpu.VMEM((2, CH), jnp.int32),
        pltpu.VMEM((CH, D), jnp.float32),
        pltpu.VMEM((2, CH), jnp.int32),
        pltpu.VMEM((2, CH), jnp.int32),
        pltpu.VMEM((N_PAD,), jnp.float32),
        pltpu.SemaphoreType.DMA,
        pltpu.SemaphoreType.DMA,
        pltpu.SemaphoreType.DMA,
    ],
)
def _sc_aggregate(x, edge_index, part_out, dstage_out, *scratch):
    _sc_body(x, edge_index, part_out, dstage_out, *scratch)


ROWS_BLK = 2048
GRID = N_PAD // ROWS_BLK


def _tc_body(x_ref, p_ref, dg_ref, ws_ref, wn_ref, bs_ref, bn_ref, o_ref):
    dsum = jnp.sum(dg_ref[...], axis=0, keepdims=True)
    inv_row = 1.0 / jnp.maximum(dsum, 1.0)
    inv_col = jnp.reshape(inv_row, (ROWS_BLK, 1))
    psum = p_ref[0] + p_ref[1]
    dn = (((1,), (1,)), ((), ()))
    o_ref[...] = (
        lax.dot_general(x_ref[...], ws_ref[...], dn,
                        preferred_element_type=jnp.float32)
        + lax.dot_general(psum, wn_ref[...], dn,
                          preferred_element_type=jnp.float32) * inv_col
        + bs_ref[...] + bn_ref[...]
    )


_tc_combine = pl.pallas_call(
    _tc_body,
    grid=(GRID,),
    in_specs=[
        pl.BlockSpec((ROWS_BLK, D), lambda i: (i, 0)),
        pl.BlockSpec((NC, ROWS_BLK, D), lambda i: (0, i, 0)),
        pl.BlockSpec((NW, ROWS_BLK), lambda i: (0, i)),
        pl.BlockSpec((D, D), lambda i: (0, 0)),
        pl.BlockSpec((D, D), lambda i: (0, 0)),
        pl.BlockSpec((1, D), lambda i: (0, 0)),
        pl.BlockSpec((1, D), lambda i: (0, 0)),
    ],
    out_specs=pl.BlockSpec((ROWS_BLK, D), lambda i: (i, 0)),
    out_shape=jax.ShapeDtypeStruct((N_NODES, D), jnp.float32),
)


def kernel(x, edge_index, W_self, b_self, W_neigh, b_neigh):
    parts, dstage = _sc_aggregate(x, edge_index)
    out = _tc_combine(
        x, parts, dstage.reshape(NW, N_PAD),
        W_self, W_neigh,
        b_self[None, :], b_neigh[None, :],
    )
    return out


# speedup vs baseline: 1.1153x; 1.0001x over previous
import functools

import jax
import jax.numpy as jnp
from jax import lax
from jax.experimental import pallas as pl
from jax.experimental.pallas import tpu as pltpu
from jax.experimental.pallas import tpu_sc as plsc

N_NODES = 10000
N_EDGES = 320000
D = 128

NC = 2
NS = 16
NW = NC * NS

CH = 128
NCHUNKS = N_EDGES // CH
NCH = NCHUNKS // NW
NXTRA = NCHUNKS - NCH * NW
N_PAD = 10240
ROWS_PER_TILE = N_PAD // NS

PIPE_ITERS = 12


def _sc_body(x_hbm, ei_hbm, part_hbm, dstage_hbm,
             accum_sp,
             ix0, r0, ix1, r1, ix2,
             ixr_v, dloc_v,
             sem_i, sem_g, sem_s):
    c = lax.axis_index("c")
    s = lax.axis_index("s")
    wid = c * NS + s

    zero16 = jnp.zeros((16,), jnp.float32)
    one16 = jnp.ones((16,), jnp.float32)

    def zero_r0(i, carry):
        for k in range(8):
            r0[i, pl.ds(k * 16, 16)] = zero16
        return carry
    lax.fori_loop(0, CH, zero_r0, 0)

    for k in range(5):
        off = pl.multiple_of(s * ROWS_PER_TILE + k * 128, 8)
        pltpu.sync_copy(r0, accum_sp.at[pl.ds(off, 128)])

    def zero_dloc(i, carry):
        dloc_v[pl.ds(pl.multiple_of(i * 16, 16), 16)] = zero16
        return carry
    lax.fori_loop(0, N_PAD // 16, zero_dloc, 0)

    plsc.subcore_barrier()

    def chunk_off(j):
        return pl.multiple_of((wid + NW * j) * CH, 128)

    def start_idx(j, ix):
        pltpu.async_copy(ei_hbm.at[:, pl.ds(chunk_off(j), CH)], ix, sem_i)

    def wait_idx(ix):
        pltpu.make_async_copy(ei_hbm.at[:, pl.ds(0, CH)], ix, sem_i).wait()

    def start_gather(ix, r):
        pltpu.async_copy(x_hbm.at[ix.at[0]], r, sem_g)

    def wait_gather(r):
        pltpu.make_async_copy(x_hbm.at[pl.ds(0, CH)], r, sem_g).wait()

    def start_scatter(r, ix):
        pltpu.async_copy(r, accum_sp.at[ix.at[1]], sem_s, add=True)

    def wait_scatter(r, ix):
        pltpu.make_async_copy(r, accum_sp.at[ix.at[1]], sem_s).wait()

    def hist(ix):
        for k in range(8):
            idx16 = ix[1, pl.ds(k * 16, 16)]
            plsc.addupdate_scatter(dloc_v, [idx16], one16)

    R = (r0, r1)
    I = (ix0, ix1, ix2)

    def step(j, pr, pi, wait_scat=True, start_nxt=True, idx2=True):
        rcur, rnxt = R[pr], R[1 - pr]
        icur, inxt, inxt2 = I[pi], I[(pi + 1) % 3], I[(pi + 2) % 3]
        wait_gather(rcur)
        if start_nxt:
            wait_idx(inxt)
        if wait_scat:
            wait_scatter(rnxt, inxt2)
        if start_nxt:
            start_gather(inxt, rnxt)
        start_scatter(rcur, icur)
        if idx2:
            start_idx(j + 2, inxt2)
        hist(icur)

    start_idx(0, ix0)
    wait_idx(ix0)
    start_gather(ix0, r0)
    start_idx(1, ix1)
    step(0, 0, 0, wait_scat=False)

    def pipe(m, carry):
        j = 6 * m + 1
        step(j + 0, 1, 1)
        step(j + 1, 0, 2)
        step(j + 2, 1, 0)
        step(j + 3, 0, 1)
        step(j + 4, 1, 2)
        step(j + 5, 0, 0)
        return carry
    lax.fori_loop(0, PIPE_ITERS, pipe, 0)

    step(73, 1, 1)
    step(74, 0, 2)
    step(75, 1, 0)
    step(76, 0, 1, idx2=False)
    step(77, 1, 2, start_nxt=False, idx2=False)
    wait_scatter(r1, ix2)

    @pl.when(wid < NXTRA)
    def _():
        offr = pl.multiple_of((NCH * NW + wid) * CH, 128)
        pltpu.sync_copy(ei_hbm.at[:, pl.ds(offr, CH)], ixr_v)
        pltpu.async_copy(x_hbm.at[ixr_v.at[0]], r0, sem_g).wait()
        pltpu.sync_copy(r0, accum_sp.at[ixr_v.at[1]], add=True)
        for k in range(8):
            idx16 = ixr_v[1, pl.ds(k * 16, 16)]
            plsc.addupdate_scatter(dloc_v, [idx16], one16)

    doff = pl.multiple_of(wid * N_PAD, 8)
    pltpu.sync_copy(dloc_v, dstage_hbm.at[pl.ds(doff, N_PAD)])

    plsc.subcore_barrier()

    off = pl.multiple_of(s * ROWS_PER_TILE, 8)
    pltpu.sync_copy(accum_sp.at[pl.ds(off, ROWS_PER_TILE)],
                    part_hbm.at[c, pl.ds(off, ROWS_PER_TILE)])


@functools.partial(
    pl.kernel,
    out_type=[
        jax.ShapeDtypeStruct((NC, N_PAD, D), jnp.float32),
        jax.ShapeDtypeStruct((NW * N_PAD,), jnp.float32),
    ],
    mesh=plsc.VectorSubcoreMesh(core_axis_name="c", subcore_axis_name="s",
                                num_cores=NC),
    compiler_params=pltpu.CompilerParams(needs_layout_passes=False),
    scratch_types=[
        pltpu.VMEM_SHARED((N_PAD, D), jnp.float32),
        pltpu.VMEM((2, CH), jnp.int32),
        pltpu.VMEM((CH, D), jnp.float32),
        pltpu.VMEM((2, CH), jnp.int32),
        pltpu.VMEM((CH, D), jnp.float32),
        pltpu.VMEM((2, CH), jnp.int32),
        pltpu.VMEM((2, CH), jnp.int32),
        pltpu.VMEM((N_PAD,), jnp.float32),
        pltpu.SemaphoreType.DMA,
        pltpu.SemaphoreType.DMA,
        pltpu.SemaphoreType.DMA,
    ],
)
def _sc_aggregate(x, edge_index, part_out, dstage_out, *scratch):
    _sc_body(x, edge_index, part_out, dstage_out, *scratch)


ROWS_BLK = 2048
GRID = N_PAD // ROWS_BLK


def _tc_self_body(x_ref, ws_ref, bs_ref, bn_ref, o_ref):
    dn = (((1,), (1,)), ((), ()))
    o_ref[...] = (
        lax.dot_general(x_ref[...], ws_ref[...], dn,
                        preferred_element_type=jnp.float32)
        + bs_ref[...] + bn_ref[...]
    )


def _tc_body(self_ref, p_ref, dg_ref, wn_ref, o_ref):
    dsum = jnp.sum(dg_ref[...], axis=0, keepdims=True)
    inv_row = 1.0 / jnp.maximum(dsum, 1.0)
    inv_col = jnp.reshape(inv_row, (ROWS_BLK, 1))
    psum = p_ref[0] + p_ref[1]
    dn = (((1,), (1,)), ((), ()))
    o_ref[...] = (
        self_ref[...]
        + lax.dot_general(psum, wn_ref[...], dn,
                          preferred_element_type=jnp.float32) * inv_col
    )


_tc_self = pl.pallas_call(
    _tc_self_body,
    grid=(GRID,),
    in_specs=[
        pl.BlockSpec((ROWS_BLK, D), lambda i: (i, 0)),
        pl.BlockSpec((D, D), lambda i: (0, 0)),
        pl.BlockSpec((1, D), lambda i: (0, 0)),
        pl.BlockSpec((1, D), lambda i: (0, 0)),
    ],
    out_specs=pl.BlockSpec((ROWS_BLK, D), lambda i: (i, 0)),
    out_shape=jax.ShapeDtypeStruct((N_NODES, D), jnp.float32),
)

_tc_combine = pl.pallas_call(
    _tc_body,
    grid=(GRID,),
    in_specs=[
        pl.BlockSpec((ROWS_BLK, D), lambda i: (i, 0)),
        pl.BlockSpec((NC, ROWS_BLK, D), lambda i: (0, i, 0)),
        pl.BlockSpec((NW, ROWS_BLK), lambda i: (0, i)),
        pl.BlockSpec((D, D), lambda i: (0, 0)),
    ],
    out_specs=pl.BlockSpec((ROWS_BLK, D), lambda i: (i, 0)),
    out_shape=jax.ShapeDtypeStruct((N_NODES, D), jnp.float32),
)


def kernel(x, edge_index, W_self, b_self, W_neigh, b_neigh):
    parts, dstage = _sc_aggregate(x, edge_index)
    selfpart = _tc_self(x, W_self, b_self[None, :], b_neigh[None, :])
    out = _tc_combine(selfpart, parts, dstage.reshape(NW, N_PAD), W_neigh)
    return out
